# baseline (speedup 1.0000x reference)
import numpy as np

import jax
import jax.numpy as jnp
from jax import lax
from jax.experimental import pallas as pl
from jax.experimental.pallas import tpu as pltpu
from jax.experimental.pallas import tpu_sc as plsc

NUM_EMB = 1000000
FEAT = 32
BATCH = 16384
FIELDS = 26

NC = 2
NS = 16
NW = NC * NS

TBLK = 512
NBLK = -(-NUM_EMB // TBLK)
STAGE_ROWS = NUM_EMB // 4

_SEL = np.zeros((4, TBLK // 4, TBLK), np.float32)
for _j in range(4):
    _r = np.arange(TBLK // 4)
    _SEL[_j, _r, 4 * _r + _j] = 1.0


def _tc_tr_body(sel_ref, x_ref, out_ref):
    x = x_ref[...]
    for j in range(4):
        out_ref[:, pl.ds(FEAT * j, FEAT)] = lax.dot_general(
            sel_ref[j], x, (((1,), (1,)), ((), ())),
            preferred_element_type=jnp.float32)


def _tc_transpose(tt):
    return pl.pallas_call(
        _tc_tr_body,
        grid=(NBLK,),
        in_specs=[
            pl.BlockSpec((4, TBLK // 4, TBLK), lambda i: (0, 0, 0)),
            pl.BlockSpec((FEAT, TBLK), lambda i: (0, i)),
        ],
        out_specs=pl.BlockSpec((TBLK // 4, 128), lambda i: (i, 0)),
        out_shape=jax.ShapeDtypeStruct((STAGE_ROWS, 128), jnp.float32),
    )(jnp.asarray(_SEL), tt)


B_PER_W = BATCH // NW


def _embed_body(ipt_hbm, table_hbm, out_hbm, blk_v, rows0, rows1,
                sem0, sem1):
    wid = lax.axis_index("s") * NC + lax.axis_index("c")
    b0 = wid * B_PER_W
    pltpu.sync_copy(ipt_hbm.at[:, pl.ds(b0, B_PER_W)], blk_v)
    bufs = (rows0, rows1)
    sems = (sem0, sem1)
    pltpu.async_copy(table_hbm.at[blk_v.at[0]], bufs[0], sems[0])

    def _field_pair(g, carry):
        for b in (0, 1):
            fl = 2 * g + b
            p, q = b, 1 - b

            @pl.when(fl + 1 < FIELDS)
            def _():
                pltpu.async_copy(
                    table_hbm.at[blk_v.at[fl + 1]], bufs[q], sems[q])

            pltpu.make_async_copy(
                table_hbm.at[blk_v.at[fl]], bufs[p], sems[p]).wait()
            pltpu.sync_copy(bufs[p], out_hbm.at[pl.ds(b0, B_PER_W), fl, :])
        return carry

    lax.fori_loop(0, FIELDS // 2, _field_pair, 0)


_embed_call = pl.kernel(
    _embed_body,
    mesh=plsc.VectorSubcoreMesh(core_axis_name="c", subcore_axis_name="s"),
    out_type=jax.ShapeDtypeStruct((BATCH, FIELDS, FEAT), jnp.float32),
    scratch_types=[
        pltpu.VMEM((FIELDS, B_PER_W), jnp.int32),
        pltpu.VMEM((B_PER_W, FEAT), jnp.float32),
        pltpu.VMEM((B_PER_W, FEAT), jnp.float32),
        pltpu.SemaphoreType.DMA,
        pltpu.SemaphoreType.DMA,
    ],
    compiler_params=pltpu.CompilerParams(use_tc_tiling_on_sc=False),
)


def kernel(ip, table):
    staged = _tc_transpose(table.T)
    table_rm = staged.reshape(NUM_EMB, FEAT)
    return _embed_call(ip.T, table_rm)


# speedup vs baseline: 2.3359x; 2.3359x over previous
import numpy as np

import jax
import jax.numpy as jnp
from jax import lax
from jax.experimental import pallas as pl
from jax.experimental.pallas import tpu as pltpu
from jax.experimental.pallas import tpu_sc as plsc

NUM_EMB = 1000000
FEAT = 32
BATCH = 16384
FIELDS = 26

NC = 2
NS = 16
NW = NC * NS
L = 16

GCOLS = 6
GLANES = GCOLS * 128
GROWS = GLANES // 4
NGROUPS = (NUM_EMB // 128) // GCOLS
BASE_G = NGROUPS // NW
EXTRA_G = NGROUPS % NW
TAIL_E = NUM_EMB - NGROUPS * GLANES
STAGE_ROWS = NUM_EMB // 4


def _tr_body(tt_hbm, rm_hbm, in0, in1, ob0, ob1, tin, tob,
             si0, si1, so0, so1):
    wid = lax.axis_index("s") * NC + lax.axis_index("c")
    cnt = BASE_G + (wid < EXTRA_G).astype(jnp.int32)

    lanes = lax.iota(jnp.int32, L)
    lpat = [lax.bitwise_and(lanes + d, L - 1) for d in range(L)]
    rpat = [lax.shift_right_logical(lpat[d], 2) for d in range(L)]
    cpat = [lax.shift_left(lax.bitwise_and(lpat[d], 3), 5) + lanes
            for d in range(L)]

    def _lane0(k):
        return pl.multiple_of((wid + NW * k) * GLANES, 128)

    def _transpose(src, dst, nlanes):
        @plsc.parallel_loop(0, nlanes // L, unroll=2)
        def _sub(m):
            l0 = m * L
            r0 = m * 4
            for f0 in (0, 16):
                for d in range(L):
                    vals = plsc.load_gather(src, [lanes + f0, lpat[d] + l0])
                    plsc.store_scatter(dst, [rpat[d] + r0, cpat[d] + f0],
                                       vals)

    def _step(k, cur_in, cur_si, cur_ob, cur_so, nxt_in, nxt_si):
        g = wid + NW * k

        @pl.when(k + 1 < cnt)
        def _():
            pltpu.async_copy(
                tt_hbm.at[:, pl.ds(_lane0(k + 1), GLANES)], nxt_in, nxt_si)

        pltpu.make_async_copy(
            tt_hbm.at[:, pl.ds(0, GLANES)], cur_in, cur_si).wait()

        @pl.when(k >= 2)
        def _():
            pltpu.make_async_copy(
                cur_ob, rm_hbm.at[pl.ds(0, GROWS), :], cur_so).wait()

        _transpose(cur_in, cur_ob, GLANES)
        pltpu.async_copy(cur_ob, rm_hbm.at[pl.ds(g * GROWS, GROWS), :],
                         cur_so)

    pltpu.async_copy(tt_hbm.at[:, pl.ds(_lane0(0), GLANES)], in0, si0)

    def _pair(k, carry):
        @pl.when(lax.bitwise_and(k, 1) == 0)
        def _():
            _step(k, in0, si0, ob0, so0, in1, si1)

        @pl.when(lax.bitwise_and(k, 1) == 1)
        def _():
            _step(k, in1, si1, ob1, so1, in0, si0)
        return carry

    lax.fori_loop(0, cnt, _pair, 0)
    pltpu.make_async_copy(ob0, rm_hbm.at[pl.ds(0, GROWS), :], so0).wait()
    pltpu.make_async_copy(ob1, rm_hbm.at[pl.ds(0, GROWS), :], so1).wait()

    @pl.when(wid == 0)
    def _():
        pltpu.sync_copy(tt_hbm.at[:, pl.ds(NGROUPS * GLANES, TAIL_E)], tin)
        _transpose(tin, tob, TAIL_E)
        pltpu.sync_copy(tob, rm_hbm.at[pl.ds(NGROUPS * GROWS, TAIL_E // 4), :])


_tr_call = pl.kernel(
    _tr_body,
    mesh=plsc.VectorSubcoreMesh(core_axis_name="c", subcore_axis_name="s"),
    out_type=jax.ShapeDtypeStruct((STAGE_ROWS, 128), jnp.float32),
    scratch_types=[
        pltpu.VMEM((FEAT, GLANES), jnp.float32),
        pltpu.VMEM((FEAT, GLANES), jnp.float32),
        pltpu.VMEM((GROWS, 128), jnp.float32),
        pltpu.VMEM((GROWS, 128), jnp.float32),
        pltpu.VMEM((FEAT, TAIL_E), jnp.float32),
        pltpu.VMEM((TAIL_E // 4, 128), jnp.float32),
        pltpu.SemaphoreType.DMA,
        pltpu.SemaphoreType.DMA,
        pltpu.SemaphoreType.DMA,
        pltpu.SemaphoreType.DMA,
    ],
    compiler_params=pltpu.CompilerParams(needs_layout_passes=False),
)

B_PER_W = BATCH // NW


def _embed_body(ipt_hbm, table_hbm, out_hbm, blk_v, rows0, rows1,
                sem0, sem1):
    wid = lax.axis_index("s") * NC + lax.axis_index("c")
    b0 = wid * B_PER_W
    pltpu.sync_copy(ipt_hbm.at[:, pl.ds(b0, B_PER_W)], blk_v)
    bufs = (rows0, rows1)
    sems = (sem0, sem1)
    pltpu.async_copy(table_hbm.at[blk_v.at[0]], bufs[0], sems[0])

    def _field_pair(g, carry):
        for b in (0, 1):
            fl = 2 * g + b
            p, q = b, 1 - b

            @pl.when(fl + 1 < FIELDS)
            def _():
                pltpu.async_copy(
                    table_hbm.at[blk_v.at[fl + 1]], bufs[q], sems[q])

            pltpu.make_async_copy(
                table_hbm.at[blk_v.at[fl]], bufs[p], sems[p]).wait()
            pltpu.sync_copy(bufs[p], out_hbm.at[pl.ds(b0, B_PER_W), fl, :])
        return carry

    lax.fori_loop(0, FIELDS // 2, _field_pair, 0)


_embed_call = pl.kernel(
    _embed_body,
    mesh=plsc.VectorSubcoreMesh(core_axis_name="c", subcore_axis_name="s"),
    out_type=jax.ShapeDtypeStruct((BATCH, FIELDS, FEAT), jnp.float32),
    scratch_types=[
        pltpu.VMEM((FIELDS, B_PER_W), jnp.int32),
        pltpu.VMEM((B_PER_W, FEAT), jnp.float32),
        pltpu.VMEM((B_PER_W, FEAT), jnp.float32),
        pltpu.SemaphoreType.DMA,
        pltpu.SemaphoreType.DMA,
    ],
    compiler_params=pltpu.CompilerParams(use_tc_tiling_on_sc=False),
)


def kernel(ip, table):
    staged = _tr_call(table.T)
    table_rm = staged.reshape(NUM_EMB, FEAT)
    return _embed_call(ip.T, table_rm)
